# baseline (speedup 1.0000x reference)
import jax
import jax.numpy as jnp
from jax.experimental import pallas as pl
from jax.experimental.pallas import tpu as pltpu

_B, _N, _NFEAT, _NH1, _NH2, _NCLASS = 2, 2048, 128, 512, 256, 40
_BANDW = 10
_RB = 512
_WIN = 768
_T = _N // _RB


def _band_mask(r0, c0, rows, cols):
    ri = jax.lax.broadcasted_iota(jnp.int32, (rows, cols), 0)
    ci = jax.lax.broadcasted_iota(jnp.int32, (rows, cols), 1)
    delta = (r0 + ri) - (c0 + ci)
    return (jnp.abs(delta) <= _BANDW).astype(jnp.float32)


def _dot(a, b):
    return jnp.dot(a, b, preferred_element_type=jnp.float32)


def _body(adj_ref, x_ref, W1_ref, b1_ref, Wb1_ref, bb1_ref,
          W3c_ref, b3_ref, bb3_ref, Wfc_ref, bfc_ref,
          out_ref, adjs_ref, G_ref, Gb_ref, acc_ref):
    b = pl.program_id(0)
    p = pl.program_id(1)
    i = pl.program_id(2)
    r0 = i * _RB
    c0 = jnp.clip(i * (_RB // 128) - (_WIN - _RB) // 256, 0, (_N - _WIN) // 128) * 128

    @pl.when(p == 0)
    def _layer1():
        adj_tile = adj_ref[0]
        adjs_ref[pl.ds(r0, _RB), :] = adj_tile
        ax = _dot(adj_tile, x_ref[0])
        aw = adj_ref[0, :, pl.ds(c0, _WIN)]
        m = _band_mask(r0, c0, _RB, _WIN)
        bx = _dot(aw * m, x_ref[0, pl.ds(c0, _WIN), :])
        h = (jax.nn.relu(_dot(ax, W1_ref[:]) + b1_ref[:])
             + jax.nn.relu(_dot(bx, Wb1_ref[:]) + bb1_ref[:]))
        Gcat = _dot(h, W3c_ref[:])
        G_ref[pl.ds(r0, _RB), :] = Gcat[:, : _NH2]
        Gb_ref[pl.ds(r0, _RB), :] = Gcat[:, _NH2 :]

    @pl.when(p == 1)
    def _layer2():
        adj_tile = adjs_ref[pl.ds(r0, _RB), :]
        nl = jax.nn.relu(_dot(adj_tile, G_ref[:]) + b3_ref[:])
        aw = adjs_ref[pl.ds(r0, _RB), pl.ds(c0, _WIN)]
        m = _band_mask(r0, c0, _RB, _WIN)
        lc = jax.nn.relu(_dot(aw * m, Gb_ref[pl.ds(c0, _WIN), :]) + bb3_ref[:])
        h2 = nl + lc
        tile_sum = jnp.sum(h2, axis=0, keepdims=True)

        @pl.when(i == 0)
        def _():
            acc_ref[:] = jnp.zeros_like(acc_ref)

        acc_ref[:] += tile_sum

        @pl.when(i == _T - 1)
        def _():
            mean = acc_ref[:] / float(_N)
            out_ref[pl.ds(b, 1), :] = _dot(mean, Wfc_ref[:]) + bfc_ref[:]


@jax.jit
def kernel(x, adj, W1, b1, Wb1, bb1, W3, b3, Wb3, bb3, Wfc, bfc):
    b1r = b1.reshape(1, _NH1)
    bb1r = bb1.reshape(1, _NH1)
    b3r = b3.reshape(1, _NH2)
    bb3r = bb3.reshape(1, _NH2)
    bfcr = bfc.reshape(1, _NCLASS)
    W3c = jnp.concatenate([W3, Wb3], axis=1)

    full = lambda shape: pl.BlockSpec(shape, lambda b, p, i: (0,) * len(shape))

    out = pl.pallas_call(
        _body,
        grid=(_B, 2, _T),
        in_specs=[
            pl.BlockSpec((1, _RB, _N), lambda b, p, i: (b, i * (1 - p), 0)),
            pl.BlockSpec((1, _N, _NFEAT), lambda b, p, i: (b, 0, 0)),
            full((_NFEAT, _NH1)),
            full((1, _NH1)),
            full((_NFEAT, _NH1)),
            full((1, _NH1)),
            full((_NH1, 2 * _NH2)),
            full((1, _NH2)),
            full((1, _NH2)),
            full((_NH2, _NCLASS)),
            full((1, _NCLASS)),
        ],
        out_specs=pl.BlockSpec((_B, _NCLASS), lambda b, p, i: (0, 0)),
        out_shape=jax.ShapeDtypeStruct((_B, _NCLASS), jnp.float32),
        scratch_shapes=[
            pltpu.VMEM((_N, _N), jnp.float32),
            pltpu.VMEM((_N, _NH2), jnp.float32),
            pltpu.VMEM((_N, _NH2), jnp.float32),
            pltpu.VMEM((1, _NH2), jnp.float32),
        ],
    )(adj, x, W1, b1r, Wb1, bb1r, W3c, b3r, bb3r, Wfc, bfcr)

    return out


# speedup vs baseline: 1.0912x; 1.0912x over previous
import jax
import jax.numpy as jnp
from jax.experimental import pallas as pl
from jax.experimental.pallas import tpu as pltpu

_B, _N, _NFEAT, _NH1, _NH2, _NCLASS = 2, 2048, 128, 512, 256, 40
_BANDW = 10
_RB = 512
_HR = 256
_WIN = 512
_T = _N // _RB


def _band_mask(r0, c0, rows, cols):
    ri = jax.lax.broadcasted_iota(jnp.int32, (rows, cols), 0)
    ci = jax.lax.broadcasted_iota(jnp.int32, (rows, cols), 1)
    delta = (r0 + ri) - (c0 + ci)
    return (jnp.abs(delta) <= _BANDW).astype(jnp.float32)


def _dot(a, b):
    return jnp.dot(a, b, preferred_element_type=jnp.float32)


def _body(adj_ref, x_ref, W1_ref, b1_ref, Wb1_ref, bb1_ref,
          W3_ref, b3_ref, Wb3_ref, bb3_ref, Wfc_ref, bfc_ref,
          out_ref, adjs_ref, G_ref, Gb_ref, acc_ref):
    b = pl.program_id(0)
    p = pl.program_id(1)
    i = pl.program_id(2)
    r0 = i * _RB
    ch0 = jnp.clip(i * 4 - 1, 0, (_N - _WIN) // 128) * 128
    ch1 = jnp.clip(i * 4 + 1, 0, (_N - _WIN) // 128) * 128

    @pl.when(p == 0)
    def _layer1():
        adj_tile = adj_ref[0]
        adjs_ref[pl.ds(r0, _RB), :] = adj_tile
        ax = _dot(adj_tile, x_ref[0])
        bx0 = _dot(adj_ref[0, pl.ds(0, _HR), pl.ds(ch0, _WIN)]
                   * _band_mask(r0, ch0, _HR, _WIN),
                   x_ref[0, pl.ds(ch0, _WIN), :])
        bx1 = _dot(adj_ref[0, pl.ds(_HR, _HR), pl.ds(ch1, _WIN)]
                   * _band_mask(r0 + _HR, ch1, _HR, _WIN),
                   x_ref[0, pl.ds(ch1, _WIN), :])
        bx = jnp.concatenate([bx0, bx1], axis=0)
        h = (jax.nn.relu(_dot(ax, W1_ref[:]) + b1_ref[:])
             + jax.nn.relu(_dot(bx, Wb1_ref[:]) + bb1_ref[:]))
        G_ref[pl.ds(r0, _RB), :] = _dot(h, W3_ref[:])
        Gb_ref[pl.ds(r0, _RB), :] = _dot(h, Wb3_ref[:])

    @pl.when(p == 1)
    def _layer2():
        adj_tile = adjs_ref[pl.ds(r0, _RB), :]
        nl = jax.nn.relu(_dot(adj_tile, G_ref[:]) + b3_ref[:])
        lc0 = _dot(adjs_ref[pl.ds(r0, _HR), pl.ds(ch0, _WIN)]
                   * _band_mask(r0, ch0, _HR, _WIN),
                   Gb_ref[pl.ds(ch0, _WIN), :])
        lc1 = _dot(adjs_ref[pl.ds(r0 + _HR, _HR), pl.ds(ch1, _WIN)]
                   * _band_mask(r0 + _HR, ch1, _HR, _WIN),
                   Gb_ref[pl.ds(ch1, _WIN), :])
        lc = jax.nn.relu(jnp.concatenate([lc0, lc1], axis=0) + bb3_ref[:])
        h2 = nl + lc
        tile_sum = jnp.sum(h2, axis=0, keepdims=True)

        @pl.when(i == 0)
        def _():
            acc_ref[:] = jnp.zeros_like(acc_ref)

        acc_ref[:] += tile_sum

        @pl.when(i == _T - 1)
        def _():
            mean = acc_ref[:] / float(_N)
            out_ref[pl.ds(b, 1), :] = _dot(mean, Wfc_ref[:]) + bfc_ref[:]


@jax.jit
def kernel(x, adj, W1, b1, Wb1, bb1, W3, b3, Wb3, bb3, Wfc, bfc):
    b1r = b1.reshape(1, _NH1)
    bb1r = bb1.reshape(1, _NH1)
    b3r = b3.reshape(1, _NH2)
    bb3r = bb3.reshape(1, _NH2)
    bfcr = bfc.reshape(1, _NCLASS)

    full = lambda shape: pl.BlockSpec(shape, lambda b, p, i: (0,) * len(shape))

    out = pl.pallas_call(
        _body,
        grid=(_B, 2, _T),
        in_specs=[
            pl.BlockSpec((1, _RB, _N), lambda b, p, i: (b, i * (1 - p), 0)),
            pl.BlockSpec((1, _N, _NFEAT), lambda b, p, i: (b, 0, 0)),
            full((_NFEAT, _NH1)),
            full((1, _NH1)),
            full((_NFEAT, _NH1)),
            full((1, _NH1)),
            full((_NH1, _NH2)),
            full((1, _NH2)),
            full((_NH1, _NH2)),
            full((1, _NH2)),
            full((_NH2, _NCLASS)),
            full((1, _NCLASS)),
        ],
        out_specs=pl.BlockSpec((_B, _NCLASS), lambda b, p, i: (0, 0)),
        out_shape=jax.ShapeDtypeStruct((_B, _NCLASS), jnp.float32),
        scratch_shapes=[
            pltpu.VMEM((_N, _N), jnp.float32),
            pltpu.VMEM((_N, _NH2), jnp.float32),
            pltpu.VMEM((_N, _NH2), jnp.float32),
            pltpu.VMEM((1, _NH2), jnp.float32),
        ],
    )(adj, x, W1, b1r, Wb1, bb1r, W3, b3r, Wb3, bb3r, Wfc, bfcr)

    return out
